# baseline (speedup 1.0000x reference)
import functools

import jax
import jax.numpy as jnp
from jax import lax
from jax.experimental import pallas as pl
from jax.experimental.pallas import tpu as pltpu
from jax.experimental.pallas import tpu_sc as plsc

H = 512
W = 512
NPIX = H * W
NIMG = 96
NWORK = 32
PPW = NPIX // NWORK
CHUNK = 64
NCHUNK = PPW // CHUNK
LANES = 16
IMG_GROUPS = NIMG // LANES
PK = 4 * CHUNK

RC = 256
BOX0 = 224
BOXW = 64
NBOX = BOXW * BOXW

_LOG_POLAR_DISTANCE = 2.0


def _blend(wv, gv, osb):
    def pixgrp(pg, pcarry):
        pbase = pg * LANES
        wv0 = wv[pl.ds(0 * CHUNK + pbase, LANES)]
        wv1 = wv[pl.ds(1 * CHUNK + pbase, LANES)]
        wv2 = wv[pl.ds(2 * CHUNK + pbase, LANES)]
        wv3 = wv[pl.ds(3 * CHUNK + pbase, LANES)]
        for j in range(LANES):
            p = pbase + j
            a0 = jnp.full((LANES,), wv0[j], jnp.float32)
            a1 = jnp.full((LANES,), wv1[j], jnp.float32)
            a2 = jnp.full((LANES,), wv2[j], jnp.float32)
            a3 = jnp.full((LANES,), wv3[j], jnp.float32)
            for g in range(IMG_GROUPS):
                ls = pl.ds(g * LANES, LANES)
                osb[p, ls] = (a0 * gv[0 * CHUNK + p, ls]
                              + a1 * gv[1 * CHUNK + p, ls]
                              + a2 * gv[2 * CHUNK + p, ls]
                              + a3 * gv[3 * CHUNK + p, ls])
        return pcarry

    lax.fori_loop(0, CHUNK // LANES, pixgrp, 0, unroll=False)


def _sc_body(packi, packw, dataT, tabc, out,
             pi0, pi1, pi2, pi3, pw0, pw1, pw2, pw3,
             g0, g1, g2, g3, osb,
             sm0, sm1, sm2, sm3, ss0, ss1, ss2, ss3):
    c = lax.axis_index("c")
    s = lax.axis_index("s")
    wid = s * 2 + c
    cbase = wid * NCHUNK

    pi_bufs = (pi0, pi1, pi2, pi3)
    pw_bufs = (pw0, pw1, pw2, pw3)
    g_bufs = (g0, g1, g2, g3)
    sems = (sm0, sm1, sm2, sm3)
    ssems = (ss0, ss1, ss2, ss3)
    NBUF = 4
    CPT = RC // CHUNK
    CPR = W // CHUNK

    def stage_fire(buf, ci):
        gid = cbase + ci
        gofs = pl.multiple_of(gid * PK, PK)
        iv, wv, gv = pi_bufs[buf], pw_bufs[buf], g_bufs[buf]
        pltpu.async_copy(packi.at[pl.ds(gofs, PK)], iv, ssems[buf])
        pltpu.async_copy(packw.at[pl.ds(gofs, PK)], wv, ssems[buf])
        pltpu.make_async_copy(packi.at[pl.ds(gofs, PK)], iv,
                              ssems[buf]).wait()
        pltpu.make_async_copy(packw.at[pl.ds(gofs, PK)], wv,
                              ssems[buf]).wait()
        central = lax.rem(gid, CPR) < CPT

        @pl.when(central)
        def _():
            pltpu.async_copy(tabc.at[iv], gv, sems[buf])

        @pl.when(jnp.logical_not(central))
        def _():
            pltpu.async_copy(dataT.at[iv], gv, sems[buf])

    def drain(buf):
        iv, gv, sem = pi_bufs[buf], g_bufs[buf], sems[buf]
        pltpu.make_async_copy(dataT.at[iv], gv, sem).wait()

    def write_out(ci):
        start = pl.multiple_of((cbase + ci) * CHUNK, CHUNK)
        pltpu.sync_copy(osb, out.at[pl.ds(start, CHUNK)])

    for b in range(NBUF):
        stage_fire(b, b)

    def quad_body(i, carry):
        c0 = i * NBUF
        for b in range(NBUF):
            ci = c0 + b
            drain(b)
            _blend(pw_bufs[b], g_bufs[b], osb)
            write_out(ci)

            @pl.when(ci + NBUF < NCHUNK)
            def _():
                stage_fire(b, ci + NBUF)

        return carry

    lax.fori_loop(0, NCHUNK // NBUF, quad_body, 0, unroll=False)


@functools.partial(jax.jit, static_argnums=())
def _run(packi, packw, dataT, tabc):
    mesh = plsc.VectorSubcoreMesh(core_axis_name="c", subcore_axis_name="s")
    f = pl.kernel(
        _sc_body,
        mesh=mesh,
        compiler_params=pltpu.CompilerParams(use_tc_tiling_on_sc=False),
        out_type=jax.ShapeDtypeStruct((NPIX, NIMG), jnp.float32),
        scratch_types=(
            [pltpu.VMEM((PK,), jnp.int32) for _ in range(4)]
            + [pltpu.VMEM((PK,), jnp.float32) for _ in range(4)]
            + [pltpu.VMEM((PK, NIMG), jnp.float32) for _ in range(4)]
            + [pltpu.VMEM((CHUNK, NIMG), jnp.float32)]
            + [pltpu.SemaphoreType.DMA for _ in range(8)]
        ),
    )
    return f(packi, packw, dataT, tabc)


def _grid():
    max_r = jnp.log(
        jnp.linalg.norm(jnp.asarray((H, W), dtype=jnp.float32)) / 2.0
        * _LOG_POLAR_DISTANCE)
    theta, r = jnp.meshgrid(jnp.arange(H), jnp.arange(W), indexing="ij")
    theta = theta.astype(jnp.float32)
    r = r.astype(jnp.float32)
    X = jnp.exp(r * max_r / W) * jnp.cos(theta * 2.0 * jnp.pi / H)
    Y = jnp.exp(r * max_r / W) * jnp.sin(theta * 2.0 * jnp.pi / H)
    X = W / 2.0 + X
    Y = H / 2.0 - Y

    y_down = jnp.clip(Y.astype(jnp.int32), 0, H - 1)
    x_down = jnp.clip(X.astype(jnp.int32), 0, W - 1)
    y_up = jnp.clip(y_down + 1, 0, H - 1)
    x_up = jnp.clip(x_down + 1, 0, W - 1)

    dd = (Y - y_down) ** 2 + (X - x_down) ** 2
    du = (Y - y_down) ** 2 + (X - x_up) ** 2
    ud = (Y - y_up) ** 2 + (X - x_down) ** 2
    uu = (Y - y_up) ** 2 + (X - x_up) ** 2
    tot = dd + du + ud + uu

    central = (jnp.arange(W)[None, :] < RC)

    def pack_idx(yy, xx):
        full = yy * W + xx
        boxed = (yy - BOX0) * BOXW + (xx - BOX0)
        return jnp.where(central, boxed, full).reshape(-1)

    idx = jnp.stack([
        pack_idx(y_down, x_down),
        pack_idx(y_down, x_up),
        pack_idx(y_up, x_down),
        pack_idx(y_up, x_up),
    ]).astype(jnp.int32)
    wts = jnp.stack([
        (dd / tot).reshape(-1),
        (du / tot).reshape(-1),
        (ud / tot).reshape(-1),
        (uu / tot).reshape(-1),
    ])
    packi = (idx.reshape(4, NPIX // CHUNK, CHUNK)
             .transpose(1, 0, 2).reshape(-1))
    packw = (wts.reshape(4, NPIX // CHUNK, CHUNK)
             .transpose(1, 0, 2).reshape(-1))
    return packi, packw


def kernel(data):
    packi, packw = _grid()
    d3 = data.reshape(NIMG, H, W)
    dataT = d3.reshape(NIMG, NPIX).transpose(1, 0)
    tabc = (d3[:, BOX0:BOX0 + BOXW, BOX0:BOX0 + BOXW]
            .reshape(NIMG, NBOX).transpose(1, 0))
    outT = _run(packi, packw, dataT, tabc)
    return outT.transpose(1, 0).reshape(data.shape)


# speedup vs baseline: 1.0155x; 1.0155x over previous
import functools

import jax
import jax.numpy as jnp
from jax import lax
from jax.experimental import pallas as pl
from jax.experimental.pallas import tpu as pltpu
from jax.experimental.pallas import tpu_sc as plsc

H = 512
W = 512
NPIX = H * W
NIMG = 96
NWORK = 32
PPW = NPIX // NWORK
CHUNK = 128
NCHUNK = PPW // CHUNK
LANES = 16
IMG_GROUPS = NIMG // LANES
PK = 4 * CHUNK

RC = 256
BOX0 = 224
BOXW = 64
NBOX = BOXW * BOXW

_LOG_POLAR_DISTANCE = 2.0


def _blend(wv, gv, osb):
    def pixgrp(pg, pcarry):
        pbase = pg * LANES
        wv0 = wv[pl.ds(0 * CHUNK + pbase, LANES)]
        wv1 = wv[pl.ds(1 * CHUNK + pbase, LANES)]
        wv2 = wv[pl.ds(2 * CHUNK + pbase, LANES)]
        wv3 = wv[pl.ds(3 * CHUNK + pbase, LANES)]
        for j in range(LANES):
            p = pbase + j
            a0 = jnp.full((LANES,), wv0[j], jnp.float32)
            a1 = jnp.full((LANES,), wv1[j], jnp.float32)
            a2 = jnp.full((LANES,), wv2[j], jnp.float32)
            a3 = jnp.full((LANES,), wv3[j], jnp.float32)
            for g in range(IMG_GROUPS):
                ls = pl.ds(g * LANES, LANES)
                osb[p, ls] = (a0 * gv[0, p, ls] + a1 * gv[1, p, ls]
                              + a2 * gv[2, p, ls] + a3 * gv[3, p, ls])
        return pcarry

    lax.fori_loop(0, CHUNK // LANES, pixgrp, 0, unroll=False)


def _sc_body(packi, packw, dataT, tabc, out,
             pia, pib, pwa, pwb, ga, gb, osb,
             sema, semb, semsa, semsb):
    c = lax.axis_index("c")
    s = lax.axis_index("s")
    wid = s * 2 + c
    cbase = wid * NCHUNK

    pi_bufs = (pia, pib)
    pw_bufs = (pwa, pwb)
    g_bufs = (ga, gb)
    sems = (sema, semb)
    ssems = (semsa, semsb)

    def stage_fire(buf, ci):
        gid = cbase + ci
        g0 = pl.multiple_of(gid * PK, PK)
        iv, wv, gv = pi_bufs[buf], pw_bufs[buf], g_bufs[buf]
        pltpu.async_copy(packi.at[pl.ds(g0, PK)], iv, ssems[buf])
        pltpu.async_copy(packw.at[pl.ds(g0, PK)], wv, ssems[buf])
        pltpu.make_async_copy(packi.at[pl.ds(g0, PK)], iv, ssems[buf]).wait()
        pltpu.make_async_copy(packw.at[pl.ds(g0, PK)], wv, ssems[buf]).wait()
        central = lax.rem(gid, 4) < 2

        @pl.when(central)
        def _():
            for k in range(4):
                pltpu.async_copy(tabc.at[iv.at[pl.ds(k * CHUNK, CHUNK)]],
                                 gv.at[k], sems[buf])

        @pl.when(jnp.logical_not(central))
        def _():
            for k in range(4):
                pltpu.async_copy(dataT.at[iv.at[pl.ds(k * CHUNK, CHUNK)]],
                                 gv.at[k], sems[buf])

    def drain(buf):
        iv, gv, sem = pi_bufs[buf], g_bufs[buf], sems[buf]
        for k in range(4):
            pltpu.make_async_copy(dataT.at[iv.at[pl.ds(k * CHUNK, CHUNK)]],
                                  gv.at[k], sem).wait()

    def write_out(ci):
        start = pl.multiple_of((cbase + ci) * CHUNK, CHUNK)
        pltpu.sync_copy(osb, out.at[pl.ds(start, CHUNK)])

    stage_fire(0, 0)
    stage_fire(1, 1)

    def pair_body(i, carry):
        c0 = i * 2
        more = c0 + 2 < NCHUNK

        drain(0)
        _blend(pwa, ga, osb)
        write_out(c0)

        @pl.when(more)
        def _():
            stage_fire(0, c0 + 2)

        drain(1)
        _blend(pwb, gb, osb)
        write_out(c0 + 1)

        @pl.when(more)
        def _():
            stage_fire(1, c0 + 3)

        return carry

    lax.fori_loop(0, NCHUNK // 2, pair_body, 0, unroll=False)


@functools.partial(jax.jit, static_argnums=())
def _run(packi, packw, dataT, tabc):
    mesh = plsc.VectorSubcoreMesh(core_axis_name="c", subcore_axis_name="s")
    f = pl.kernel(
        _sc_body,
        mesh=mesh,
        compiler_params=pltpu.CompilerParams(use_tc_tiling_on_sc=False),
        out_type=jax.ShapeDtypeStruct((NPIX, NIMG), jnp.float32),
        scratch_types=[
            pltpu.VMEM((PK,), jnp.int32),
            pltpu.VMEM((PK,), jnp.int32),
            pltpu.VMEM((PK,), jnp.float32),
            pltpu.VMEM((PK,), jnp.float32),
            pltpu.VMEM((4, CHUNK, NIMG), jnp.float32),
            pltpu.VMEM((4, CHUNK, NIMG), jnp.float32),
            pltpu.VMEM((CHUNK, NIMG), jnp.float32),
            pltpu.SemaphoreType.DMA,
            pltpu.SemaphoreType.DMA,
            pltpu.SemaphoreType.DMA,
            pltpu.SemaphoreType.DMA,
        ],
    )
    return f(packi, packw, dataT, tabc)


def _grid():
    max_r = jnp.log(
        jnp.linalg.norm(jnp.asarray((H, W), dtype=jnp.float32)) / 2.0
        * _LOG_POLAR_DISTANCE)
    theta, r = jnp.meshgrid(jnp.arange(H), jnp.arange(W), indexing="ij")
    theta = theta.astype(jnp.float32)
    r = r.astype(jnp.float32)
    X = jnp.exp(r * max_r / W) * jnp.cos(theta * 2.0 * jnp.pi / H)
    Y = jnp.exp(r * max_r / W) * jnp.sin(theta * 2.0 * jnp.pi / H)
    X = W / 2.0 + X
    Y = H / 2.0 - Y

    y_down = jnp.clip(Y.astype(jnp.int32), 0, H - 1)
    x_down = jnp.clip(X.astype(jnp.int32), 0, W - 1)
    y_up = jnp.clip(y_down + 1, 0, H - 1)
    x_up = jnp.clip(x_down + 1, 0, W - 1)

    dd = (Y - y_down) ** 2 + (X - x_down) ** 2
    du = (Y - y_down) ** 2 + (X - x_up) ** 2
    ud = (Y - y_up) ** 2 + (X - x_down) ** 2
    uu = (Y - y_up) ** 2 + (X - x_up) ** 2
    tot = dd + du + ud + uu

    central = (jnp.arange(W)[None, :] < RC)

    def pack_idx(yy, xx):
        full = yy * W + xx
        boxed = (yy - BOX0) * BOXW + (xx - BOX0)
        return jnp.where(central, boxed, full).reshape(-1)

    idx = jnp.stack([
        pack_idx(y_down, x_down),
        pack_idx(y_down, x_up),
        pack_idx(y_up, x_down),
        pack_idx(y_up, x_up),
    ]).astype(jnp.int32)
    wts = jnp.stack([
        (dd / tot).reshape(-1),
        (du / tot).reshape(-1),
        (ud / tot).reshape(-1),
        (uu / tot).reshape(-1),
    ])
    packi = (idx.reshape(4, NPIX // CHUNK, CHUNK)
             .transpose(1, 0, 2).reshape(-1))
    packw = (wts.reshape(4, NPIX // CHUNK, CHUNK)
             .transpose(1, 0, 2).reshape(-1))
    return packi, packw


def kernel(data):
    packi, packw = _grid()
    d3 = data.reshape(NIMG, H, W)
    dataT = d3.reshape(NIMG, NPIX).transpose(1, 0)
    tabc = (d3[:, BOX0:BOX0 + BOXW, BOX0:BOX0 + BOXW]
            .reshape(NIMG, NBOX).transpose(1, 0))
    outT = _run(packi, packw, dataT, tabc)
    return outT.transpose(1, 0).reshape(data.shape)
